# Initial kernel scaffold; baseline (speedup 1.0000x reference)
import jax
import jax.numpy as jnp
from jax.experimental import pallas as pl


def kernel(input1):
    raise NotImplementedError("write your pallas kernel here")


# speedup vs baseline: 88.9456x; 88.9456x over previous
import functools

import jax
import jax.numpy as jnp
from jax import lax
from jax.experimental import pallas as pl
from jax.experimental.pallas import tpu as pltpu
from jax.experimental.pallas import tpu_sc as plsc

B, C, H, W = 8, 2, 512, 512
HW = H * W
NC, NS, L = 2, 16, 16
BPC = B // NC
ROWS_PER_TEC = H // NS
PX_PER_TEC = ROWS_PER_TEC * W
CH = 4096
NCHUNK = PX_PER_TEC // CH
SEG = 128
NSEG = CH // SEG
NB = 1024
ZCH = 4096

_mesh = plsc.VectorSubcoreMesh(core_axis_name="c", subcore_axis_name="s")


@functools.partial(
    pl.kernel,
    mesh=_mesh,
    out_type=jax.ShapeDtypeStruct((B, C, HW), jnp.float32),
    scratch_types=[
        pltpu.VMEM_SHARED((HW,), jnp.float32),
        pltpu.VMEM_SHARED((HW,), jnp.float32),
        pltpu.VMEM_SHARED((HW,), jnp.float32),
        pltpu.VMEM((CH,), jnp.float32),
        pltpu.VMEM((CH,), jnp.float32),
        pltpu.VMEM((CH,), jnp.float32),
        pltpu.VMEM((CH,), jnp.float32),
        pltpu.VMEM((CH,), jnp.float32),
        pltpu.VMEM((4, NSEG, SEG), jnp.int32),
        pltpu.VMEM((ZCH,), jnp.float32),
        pltpu.VMEM((NB,), jnp.float32),
        pltpu.VMEM((NB,), jnp.float32),
        pltpu.VMEM((NB,), jnp.float32),
        pltpu.VMEM((NB,), jnp.float32),
        pltpu.VMEM((NB,), jnp.float32),
        pltpu.SemaphoreType.DMA,
    ],
)
def _warp_kernel(in_hbm, out_hbm, accx, accy, accc, fxb, fyb, vxb, vyb, wb,
                 idx, zbuf, nbx, nby, nbc, oxb, oyb, sem):
    cid = lax.axis_index("c")
    sid = lax.axis_index("s")
    px0 = sid * PX_PER_TEC
    lanes = lax.iota(jnp.int32, L)
    zeros16 = jnp.zeros((L,), jnp.float32)
    lim = jnp.float32(W - 1)

    def zfill(k, _):
        zbuf[pl.ds(k * L, L)] = zeros16
        return 0
    lax.fori_loop(0, ZCH // L, zfill, 0)

    def do_batch(bi, _):
        b = cid * BPC + bi

        def zdma(t, _):
            base = px0 + t * ZCH
            pltpu.sync_copy(zbuf, accx.at[pl.ds(base, ZCH)])
            pltpu.sync_copy(zbuf, accy.at[pl.ds(base, ZCH)])
            pltpu.sync_copy(zbuf, accc.at[pl.ds(base, ZCH)])
            return 0
        lax.fori_loop(0, PX_PER_TEC // ZCH, zdma, 0)
        plsc.subcore_barrier()

        def chunk(ci, _):
            cbase = px0 + ci * CH
            pltpu.sync_copy(in_hbm.at[b, 0, pl.ds(cbase, CH)], fxb)
            pltpu.sync_copy(in_hbm.at[b, 1, pl.ds(cbase, CH)], fyb)

            def vec(kv, _):
                k = kv * L
                p = cbase + k + lanes
                row = lax.shift_right_logical(p, 9)
                col = lax.bitwise_and(p, W - 1)
                fxv = fxb[pl.ds(k, L)]
                fyv = fyb[pl.ds(k, L)]
                x2 = col.astype(jnp.float32) + fxv
                y2 = row.astype(jnp.float32) + fyv
                valid = ((x2 >= 0.0) & (y2 >= 0.0)
                         & (x2 <= lim) & (y2 <= lim))
                w1 = jnp.where(valid, jnp.float32(1.0), jnp.float32(0.0))
                ixl = jnp.clip(x2.astype(jnp.int32), 0, W - 1)
                iyt = jnp.clip(y2.astype(jnp.int32), 0, H - 1)
                ixr = jnp.minimum(ixl + 1, W - 1)
                iyb = jnp.minimum(iyt + 1, H - 1)
                vxb[pl.ds(k, L)] = -fxv * w1
                vyb[pl.ds(k, L)] = -fyv * w1
                wb[pl.ds(k, L)] = w1
                rt = iyt * W
                rb = iyb * W
                seg = kv // (SEG // L)
                off = (kv % (SEG // L)) * L
                idx[0, seg, pl.ds(off, L)] = rt + ixl
                idx[1, seg, pl.ds(off, L)] = rt + ixr
                idx[2, seg, pl.ds(off, L)] = rb + ixl
                idx[3, seg, pl.ds(off, L)] = rb + ixr
                return 0
            lax.fori_loop(0, CH // L, vec, 0)

            def seg_loop(j, _):
                s = j * SEG
                copies = []
                for cn in range(4):
                    copies.append(pltpu.async_copy(
                        vxb.at[pl.ds(s, SEG)], accx.at[idx.at[cn, j]], sem,
                        add=True))
                    copies.append(pltpu.async_copy(
                        vyb.at[pl.ds(s, SEG)], accy.at[idx.at[cn, j]], sem,
                        add=True))
                    copies.append(pltpu.async_copy(
                        wb.at[pl.ds(s, SEG)], accc.at[idx.at[cn, j]], sem,
                        add=True))
                for cp in copies:
                    cp.wait()
                return 0
            lax.fori_loop(0, NSEG, seg_loop, 0)
            return 0
        lax.fori_loop(0, NCHUNK, chunk, 0)
        plsc.subcore_barrier()

        def norm(g, _):
            base = px0 + g * NB
            pltpu.sync_copy(accx.at[pl.ds(base, NB)], nbx)
            pltpu.sync_copy(accy.at[pl.ds(base, NB)], nby)
            pltpu.sync_copy(accc.at[pl.ds(base, NB)], nbc)

            def nv(kv, _):
                k = kv * L
                sx = nbx[pl.ds(k, L)]
                sy = nby[pl.ds(k, L)]
                cc = nbc[pl.ds(k, L)]
                safe = jnp.where(cc > 0.0, cc, jnp.float32(1.0))
                oxb[pl.ds(k, L)] = sx / safe
                oyb[pl.ds(k, L)] = sy / safe
                return 0
            lax.fori_loop(0, NB // L, nv, 0)
            pltpu.sync_copy(oxb, out_hbm.at[b, 0, pl.ds(base, NB)])
            pltpu.sync_copy(oyb, out_hbm.at[b, 1, pl.ds(base, NB)])
            return 0
        lax.fori_loop(0, PX_PER_TEC // NB, norm, 0)
        plsc.subcore_barrier()
        return 0
    lax.fori_loop(0, BPC, do_batch, 0)


def kernel(input1):
    flat = input1.reshape(B, C, HW)
    out = _warp_kernel(flat)
    return out.reshape(B, C, H, W)


# speedup vs baseline: 99.1222x; 1.1144x over previous
import functools

import jax
import jax.numpy as jnp
from jax import lax
from jax.experimental import pallas as pl
from jax.experimental.pallas import tpu as pltpu
from jax.experimental.pallas import tpu_sc as plsc

B, C, H, W = 8, 2, 512, 512
HW = H * W
NC, NS, L = 2, 16, 16
BPC = B // NC
ROWS_PER_TEC = H // NS
PX_PER_TEC = ROWS_PER_TEC * W
CH = 4096
NCHUNK = PX_PER_TEC // CH
SEG = 128
NSEG = CH // SEG
NB = 1024
ZCH = 4096

_mesh = plsc.VectorSubcoreMesh(core_axis_name="c", subcore_axis_name="s")


@functools.partial(
    pl.kernel,
    mesh=_mesh,
    out_type=jax.ShapeDtypeStruct((B, C, HW), jnp.float32),
    scratch_types=[
        pltpu.VMEM_SHARED((HW,), jnp.float32),
        pltpu.VMEM_SHARED((HW,), jnp.float32),
        pltpu.VMEM_SHARED((HW,), jnp.float32),
        pltpu.VMEM((CH,), jnp.float32),
        pltpu.VMEM((CH,), jnp.float32),
        pltpu.VMEM((CH,), jnp.float32),
        pltpu.VMEM((CH,), jnp.float32),
        pltpu.VMEM((CH,), jnp.float32),
        pltpu.VMEM((CH,), jnp.int32),
        pltpu.VMEM((CH,), jnp.int32),
        pltpu.VMEM((CH,), jnp.int32),
        pltpu.VMEM((CH,), jnp.int32),
        pltpu.VMEM((ZCH,), jnp.float32),
        pltpu.VMEM((NB,), jnp.float32),
        pltpu.VMEM((NB,), jnp.float32),
        pltpu.VMEM((NB,), jnp.float32),
        pltpu.VMEM((NB,), jnp.float32),
        pltpu.VMEM((NB,), jnp.float32),
        pltpu.SemaphoreType.DMA,
    ],
)
def _warp_kernel(in_hbm, out_hbm, accx, accy, accc, fxb, fyb, vxb, vyb, wb,
                 idx0, idx1, idx2, idx3, zbuf, nbx, nby, nbc, oxb, oyb, sem):
    cid = lax.axis_index("c")
    sid = lax.axis_index("s")
    px0 = sid * PX_PER_TEC
    lanes = lax.iota(jnp.int32, L)
    zeros16 = jnp.zeros((L,), jnp.float32)
    lim = jnp.float32(W - 1)

    def zfill(k, _):
        zbuf[pl.ds(k * L, L)] = zeros16
        return 0
    lax.fori_loop(0, ZCH // L, zfill, 0)

    def do_batch(bi, _):
        b = cid * BPC + bi

        def zdma(t, _):
            base = px0 + t * ZCH
            pltpu.sync_copy(zbuf, accx.at[pl.ds(base, ZCH)])
            pltpu.sync_copy(zbuf, accy.at[pl.ds(base, ZCH)])
            pltpu.sync_copy(zbuf, accc.at[pl.ds(base, ZCH)])
            return 0
        lax.fori_loop(0, PX_PER_TEC // ZCH, zdma, 0)
        plsc.subcore_barrier()

        def chunk(ci, _):
            cbase = px0 + ci * CH
            pltpu.sync_copy(in_hbm.at[b, 0, pl.ds(cbase, CH)], fxb)
            pltpu.sync_copy(in_hbm.at[b, 1, pl.ds(cbase, CH)], fyb)

            def vec(kv, _):
                k = kv * L
                p = cbase + k + lanes
                row = lax.shift_right_logical(p, 9)
                col = lax.bitwise_and(p, W - 1)
                fxv = fxb[pl.ds(k, L)]
                fyv = fyb[pl.ds(k, L)]
                x2 = col.astype(jnp.float32) + fxv
                y2 = row.astype(jnp.float32) + fyv
                valid = ((x2 >= 0.0) & (y2 >= 0.0)
                         & (x2 <= lim) & (y2 <= lim))
                w1 = jnp.where(valid, jnp.float32(1.0), jnp.float32(0.0))
                ixl = jnp.clip(x2.astype(jnp.int32), 0, W - 1)
                iyt = jnp.clip(y2.astype(jnp.int32), 0, H - 1)
                ixr = jnp.minimum(ixl + 1, W - 1)
                iyb = jnp.minimum(iyt + 1, H - 1)
                vxb[pl.ds(k, L)] = -fxv * w1
                vyb[pl.ds(k, L)] = -fyv * w1
                wb[pl.ds(k, L)] = w1
                rt = iyt * W
                rb = iyb * W
                idx0[pl.ds(k, L)] = rt + ixl
                idx1[pl.ds(k, L)] = rt + ixr
                idx2[pl.ds(k, L)] = rb + ixl
                idx3[pl.ds(k, L)] = rb + ixr
                return 0
            lax.fori_loop(0, CH // L, vec, 0)

            copies = []
            for ixr_ref in (idx0, idx1, idx2, idx3):
                copies.append(pltpu.async_copy(
                    vxb, accx.at[ixr_ref], sem, add=True))
                copies.append(pltpu.async_copy(
                    vyb, accy.at[ixr_ref], sem, add=True))
                copies.append(pltpu.async_copy(
                    wb, accc.at[ixr_ref], sem, add=True))
            for cp in copies:
                cp.wait()
            return 0
        lax.fori_loop(0, NCHUNK, chunk, 0)
        plsc.subcore_barrier()

        def norm(g, _):
            base = px0 + g * NB
            pltpu.sync_copy(accx.at[pl.ds(base, NB)], nbx)
            pltpu.sync_copy(accy.at[pl.ds(base, NB)], nby)
            pltpu.sync_copy(accc.at[pl.ds(base, NB)], nbc)

            def nv(kv, _):
                k = kv * L
                sx = nbx[pl.ds(k, L)]
                sy = nby[pl.ds(k, L)]
                cc = nbc[pl.ds(k, L)]
                safe = jnp.where(cc > 0.0, cc, jnp.float32(1.0))
                oxb[pl.ds(k, L)] = sx / safe
                oyb[pl.ds(k, L)] = sy / safe
                return 0
            lax.fori_loop(0, NB // L, nv, 0)
            pltpu.sync_copy(oxb, out_hbm.at[b, 0, pl.ds(base, NB)])
            pltpu.sync_copy(oyb, out_hbm.at[b, 1, pl.ds(base, NB)])
            return 0
        lax.fori_loop(0, PX_PER_TEC // NB, norm, 0)
        plsc.subcore_barrier()
        return 0
    lax.fori_loop(0, BPC, do_batch, 0)


def kernel(input1):
    flat = input1.reshape(B, C, HW)
    out = _warp_kernel(flat)
    return out.reshape(B, C, H, W)


# speedup vs baseline: 110.2107x; 1.1119x over previous
import functools

import jax
import jax.numpy as jnp
from jax import lax
from jax.experimental import pallas as pl
from jax.experimental.pallas import tpu as pltpu
from jax.experimental.pallas import tpu_sc as plsc

B, C, H, W = 8, 2, 512, 512
HW = H * W
NC, NS, L = 2, 16, 16
BPC = B // NC
ROWS_PER_TEC = H // NS
PX_PER_TEC = ROWS_PER_TEC * W
CH = 4096
NCHUNK = PX_PER_TEC // CH
NB = 1024
ZCH = 2048

_mesh = plsc.VectorSubcoreMesh(core_axis_name="c", subcore_axis_name="s")

_chunk_bufs = [
    pltpu.VMEM((CH,), jnp.float32),
    pltpu.VMEM((CH,), jnp.float32),
]
for _ in range(2):
    _chunk_bufs += [
        pltpu.VMEM((CH,), jnp.float32),
        pltpu.VMEM((CH,), jnp.float32),
        pltpu.VMEM((CH,), jnp.float32),
        pltpu.VMEM((CH,), jnp.int32),
        pltpu.VMEM((CH,), jnp.int32),
        pltpu.VMEM((CH,), jnp.int32),
        pltpu.VMEM((CH,), jnp.int32),
    ]


@functools.partial(
    pl.kernel,
    mesh=_mesh,
    out_type=jax.ShapeDtypeStruct((B, C, HW), jnp.float32),
    scratch_types=[
        pltpu.VMEM_SHARED((HW,), jnp.float32),
        pltpu.VMEM_SHARED((HW,), jnp.float32),
        pltpu.VMEM_SHARED((HW,), jnp.float32),
        *_chunk_bufs,
        pltpu.VMEM((ZCH,), jnp.float32),
        pltpu.VMEM((NB,), jnp.float32),
        pltpu.VMEM((NB,), jnp.float32),
        pltpu.VMEM((NB,), jnp.float32),
        pltpu.VMEM((NB,), jnp.float32),
        pltpu.VMEM((NB,), jnp.float32),
        pltpu.SemaphoreType.DMA,
    ],
)
def _warp_kernel(in_hbm, out_hbm, accx, accy, accc, *rest):
    fxb, fyb = rest[0], rest[1]
    bufs = [rest[2 + 7 * p:2 + 7 * p + 7] for p in range(2)]
    zbuf, nbx, nby, nbc, oxb, oyb, sem = rest[16:]
    cid = lax.axis_index("c")
    sid = lax.axis_index("s")
    px0 = sid * PX_PER_TEC
    lanes = lax.iota(jnp.int32, L)
    zeros16 = jnp.zeros((L,), jnp.float32)
    lim = jnp.float32(W - 1)

    def zfill(k, _):
        zbuf[pl.ds(k * L, L)] = zeros16
        return 0
    lax.fori_loop(0, ZCH // L, zfill, 0)

    def compute_chunk(p, b, ci):
        vxb, vyb, wb, idx0, idx1, idx2, idx3 = bufs[p]
        cbase = px0 + ci * CH
        pltpu.sync_copy(in_hbm.at[b, 0, pl.ds(cbase, CH)], fxb)
        pltpu.sync_copy(in_hbm.at[b, 1, pl.ds(cbase, CH)], fyb)

        def vec(kv, _):
            k = kv * L
            p_ = cbase + k + lanes
            row = lax.shift_right_logical(p_, 9)
            col = lax.bitwise_and(p_, W - 1)
            fxv = fxb[pl.ds(k, L)]
            fyv = fyb[pl.ds(k, L)]
            x2 = col.astype(jnp.float32) + fxv
            y2 = row.astype(jnp.float32) + fyv
            valid = ((x2 >= 0.0) & (y2 >= 0.0)
                     & (x2 <= lim) & (y2 <= lim))
            w1 = jnp.where(valid, jnp.float32(1.0), jnp.float32(0.0))
            ixl = jnp.clip(x2.astype(jnp.int32), 0, W - 1)
            iyt = jnp.clip(y2.astype(jnp.int32), 0, H - 1)
            ixr = jnp.minimum(ixl + 1, W - 1)
            iyb = jnp.minimum(iyt + 1, H - 1)
            vxb[pl.ds(k, L)] = -fxv * w1
            vyb[pl.ds(k, L)] = -fyv * w1
            wb[pl.ds(k, L)] = w1
            rt = iyt * W
            rb = iyb * W
            idx0[pl.ds(k, L)] = rt + ixl
            idx1[pl.ds(k, L)] = rt + ixr
            idx2[pl.ds(k, L)] = rb + ixl
            idx3[pl.ds(k, L)] = rb + ixr
            return 0
        lax.fori_loop(0, CH // L, vec, 0)

    def fire_chunk(p):
        vxb, vyb, wb, idx0, idx1, idx2, idx3 = bufs[p]
        copies = []
        for ixr_ref in (idx0, idx1, idx2, idx3):
            copies.append(pltpu.async_copy(
                vxb, accx.at[ixr_ref], sem, add=True))
            copies.append(pltpu.async_copy(
                vyb, accy.at[ixr_ref], sem, add=True))
            copies.append(pltpu.async_copy(
                wb, accc.at[ixr_ref], sem, add=True))
        return copies

    def do_batch(b):
        def zdma(t, _):
            base = px0 + t * ZCH
            pltpu.sync_copy(zbuf, accx.at[pl.ds(base, ZCH)])
            pltpu.sync_copy(zbuf, accy.at[pl.ds(base, ZCH)])
            pltpu.sync_copy(zbuf, accc.at[pl.ds(base, ZCH)])
            return 0
        lax.fori_loop(0, PX_PER_TEC // ZCH, zdma, 0)
        plsc.subcore_barrier()

        inflight = None
        for ci in range(NCHUNK):
            p = ci % 2
            compute_chunk(p, b, ci)
            if inflight is not None:
                for cp in inflight:
                    cp.wait()
            inflight = fire_chunk(p)
        for cp in inflight:
            cp.wait()
        plsc.subcore_barrier()

        def norm(g, _):
            base = px0 + g * NB
            pltpu.sync_copy(accx.at[pl.ds(base, NB)], nbx)
            pltpu.sync_copy(accy.at[pl.ds(base, NB)], nby)
            pltpu.sync_copy(accc.at[pl.ds(base, NB)], nbc)

            def nv(kv, _):
                k = kv * L
                sx = nbx[pl.ds(k, L)]
                sy = nby[pl.ds(k, L)]
                cc = nbc[pl.ds(k, L)]
                r = jnp.float32(1.0) / jnp.where(cc > 0.0, cc,
                                                 jnp.float32(1.0))
                oxb[pl.ds(k, L)] = sx * r
                oyb[pl.ds(k, L)] = sy * r
                return 0
            lax.fori_loop(0, NB // L, nv, 0)
            pltpu.sync_copy(oxb, out_hbm.at[b, 0, pl.ds(base, NB)])
            pltpu.sync_copy(oyb, out_hbm.at[b, 1, pl.ds(base, NB)])
            return 0
        lax.fori_loop(0, PX_PER_TEC // NB, norm, 0)
        plsc.subcore_barrier()

    def batch_loop(bi, _):
        do_batch(cid * BPC + bi)
        return 0
    lax.fori_loop(0, BPC, batch_loop, 0)


def kernel(input1):
    flat = input1.reshape(B, C, HW)
    out = _warp_kernel(flat)
    return out.reshape(B, C, H, W)


# speedup vs baseline: 124.4566x; 1.1293x over previous
import functools

import jax
import jax.numpy as jnp
from jax import lax
from jax.experimental import pallas as pl
from jax.experimental.pallas import tpu as pltpu
from jax.experimental.pallas import tpu_sc as plsc

B, C, H, W = 8, 2, 512, 512
HW = H * W
NC, NS, L = 2, 16, 16
BPC = B // NC
ROWS_PER_TEC = H // NS
PX_PER_TEC = ROWS_PER_TEC * W
CH = 2048
NCHUNK = PX_PER_TEC // CH
NB = 2048
NG = PX_PER_TEC // NB

_mesh = plsc.VectorSubcoreMesh(core_axis_name="c", subcore_axis_name="s")

_scratch = [
    pltpu.VMEM_SHARED((HW,), jnp.float32),
    pltpu.VMEM_SHARED((HW,), jnp.float32),
    pltpu.VMEM_SHARED((HW,), jnp.float32),
    pltpu.VMEM((CH,), jnp.float32),
    pltpu.VMEM((CH,), jnp.float32),
]
for _ in range(2):
    _scratch += [
        pltpu.VMEM((CH,), jnp.float32),
        pltpu.VMEM((CH,), jnp.float32),
        pltpu.VMEM((CH,), jnp.float32),
        pltpu.VMEM((CH,), jnp.int32),
        pltpu.VMEM((CH,), jnp.int32),
        pltpu.VMEM((CH,), jnp.int32),
        pltpu.VMEM((CH,), jnp.int32),
    ]
_scratch += [pltpu.VMEM((NB,), jnp.float32)]
for _ in range(2):
    _scratch += [
        pltpu.VMEM((NB,), jnp.float32),
        pltpu.VMEM((NB,), jnp.float32),
        pltpu.VMEM((NB,), jnp.float32),
        pltpu.VMEM((NB,), jnp.float32),
        pltpu.VMEM((NB,), jnp.float32),
    ]
_scratch += [pltpu.SemaphoreType.DMA] * 6


@functools.partial(
    pl.kernel,
    mesh=_mesh,
    out_type=jax.ShapeDtypeStruct((B, C, HW), jnp.float32),
    scratch_types=_scratch,
)
def _warp_kernel(in_hbm, out_hbm, accx, accy, accc, *rest):
    fxb, fyb = rest[0], rest[1]
    cbufs = [rest[2 + 7 * p:2 + 7 * p + 7] for p in range(2)]
    zbuf = rest[16]
    nbufs = [rest[17 + 5 * p:17 + 5 * p + 5] for p in range(2)]
    sem_stream = rest[27]
    sem_load = rest[28:30]
    sem_store = rest[30:32]
    sem_zero = rest[32]
    cid = lax.axis_index("c")
    sid = lax.axis_index("s")
    px0 = sid * PX_PER_TEC
    lanes = lax.iota(jnp.int32, L)
    zeros16 = jnp.zeros((L,), jnp.float32)
    lim = jnp.float32(W - 1)

    def zfill(k, _):
        zbuf[pl.ds(k * L, L)] = zeros16
        return 0
    lax.fori_loop(0, NB // L, zfill, 0)

    def fire_zero(g):
        base = px0 + g * NB
        return [
            pltpu.async_copy(zbuf, accx.at[pl.ds(base, NB)], sem_zero),
            pltpu.async_copy(zbuf, accy.at[pl.ds(base, NB)], sem_zero),
            pltpu.async_copy(zbuf, accc.at[pl.ds(base, NB)], sem_zero),
        ]

    def compute_chunk(p, b, ci):
        vxb, vyb, wb, idx0, idx1, idx2, idx3 = cbufs[p]
        cbase = px0 + ci * CH
        pltpu.sync_copy(in_hbm.at[b, 0, pl.ds(cbase, CH)], fxb)
        pltpu.sync_copy(in_hbm.at[b, 1, pl.ds(cbase, CH)], fyb)

        def vec(kv, _):
            k = kv * L
            p_ = cbase + k + lanes
            row = lax.shift_right_logical(p_, 9)
            col = lax.bitwise_and(p_, W - 1)
            fxv = fxb[pl.ds(k, L)]
            fyv = fyb[pl.ds(k, L)]
            x2 = col.astype(jnp.float32) + fxv
            y2 = row.astype(jnp.float32) + fyv
            valid = ((x2 >= 0.0) & (y2 >= 0.0)
                     & (x2 <= lim) & (y2 <= lim))
            w1 = jnp.where(valid, jnp.float32(1.0), jnp.float32(0.0))
            ixl = jnp.clip(x2.astype(jnp.int32), 0, W - 1)
            iyt = jnp.clip(y2.astype(jnp.int32), 0, H - 1)
            ixr = jnp.minimum(ixl + 1, W - 1)
            iyb = jnp.minimum(iyt + 1, H - 1)
            vxb[pl.ds(k, L)] = -fxv * w1
            vyb[pl.ds(k, L)] = -fyv * w1
            wb[pl.ds(k, L)] = w1
            rt = iyt * W
            rb = iyb * W
            idx0[pl.ds(k, L)] = rt + ixl
            idx1[pl.ds(k, L)] = rt + ixr
            idx2[pl.ds(k, L)] = rb + ixl
            idx3[pl.ds(k, L)] = rb + ixr
            return 0
        lax.fori_loop(0, CH // L, vec, 0)

    def fire_chunk(p):
        vxb, vyb, wb, idx0, idx1, idx2, idx3 = cbufs[p]
        copies = []
        for ixr_ref in (idx0, idx1, idx2, idx3):
            copies.append(pltpu.async_copy(
                vxb, accx.at[ixr_ref], sem_stream, add=True))
            copies.append(pltpu.async_copy(
                vyb, accy.at[ixr_ref], sem_stream, add=True))
            copies.append(pltpu.async_copy(
                wb, accc.at[ixr_ref], sem_stream, add=True))
        return copies

    def fire_load(g, p):
        base = px0 + g * NB
        nbx, nby, nbc, _, _ = nbufs[p]
        return [
            pltpu.async_copy(accx.at[pl.ds(base, NB)], nbx, sem_load[p]),
            pltpu.async_copy(accy.at[pl.ds(base, NB)], nby, sem_load[p]),
            pltpu.async_copy(accc.at[pl.ds(base, NB)], nbc, sem_load[p]),
        ]

    def do_batch(b):
        inflight = None
        for ci in range(NCHUNK):
            p = ci % 2
            compute_chunk(p, b, ci)
            if inflight is not None:
                for cp in inflight:
                    cp.wait()
            inflight = fire_chunk(p)
        for cp in inflight:
            cp.wait()
        plsc.subcore_barrier()

        zero_copies = []
        store_copies = [None, None]
        loads = [None, None]
        loads[0] = fire_load(0, 0)
        for g in range(NG):
            p = g % 2
            if g + 1 < NG:
                loads[1 - p] = fire_load(g + 1, 1 - p)
            for cp in loads[p]:
                cp.wait()
            zero_copies += fire_zero(g)
            nbx, nby, nbc, oxb, oyb = nbufs[p]
            if store_copies[p] is not None:
                for cp in store_copies[p]:
                    cp.wait()

            def nv(kv, _):
                k = kv * L
                sx = nbx[pl.ds(k, L)]
                sy = nby[pl.ds(k, L)]
                cc = nbc[pl.ds(k, L)]
                r = jnp.float32(1.0) / jnp.where(cc > 0.0, cc,
                                                 jnp.float32(1.0))
                oxb[pl.ds(k, L)] = sx * r
                oyb[pl.ds(k, L)] = sy * r
                return 0
            lax.fori_loop(0, NB // L, nv, 0)
            base = px0 + g * NB
            store_copies[p] = [
                pltpu.async_copy(oxb, out_hbm.at[b, 0, pl.ds(base, NB)],
                                 sem_store[p]),
                pltpu.async_copy(oyb, out_hbm.at[b, 1, pl.ds(base, NB)],
                                 sem_store[p]),
            ]
        for sc_list in store_copies:
            if sc_list is not None:
                for cp in sc_list:
                    cp.wait()
        for cp in zero_copies:
            cp.wait()
        plsc.subcore_barrier()

    prol = []
    for g in range(NG):
        prol += fire_zero(g)
    for cp in prol:
        cp.wait()
    plsc.subcore_barrier()

    def batch_loop(bi, _):
        do_batch(cid * BPC + bi)
        return 0
    lax.fori_loop(0, BPC, batch_loop, 0)


def kernel(input1):
    flat = input1.reshape(B, C, HW)
    out = _warp_kernel(flat)
    return out.reshape(B, C, H, W)


# speedup vs baseline: 125.0648x; 1.0049x over previous
import functools

import jax
import jax.numpy as jnp
from jax import lax
from jax.experimental import pallas as pl
from jax.experimental.pallas import tpu as pltpu
from jax.experimental.pallas import tpu_sc as plsc

B, C, H, W = 8, 2, 512, 512
HW = H * W
NC, NS, L = 2, 16, 16
BPC = B // NC
ROWS_PER_TEC = H // NS
PX_PER_TEC = ROWS_PER_TEC * W
CH = 2048
NCHUNK = PX_PER_TEC // CH
NB = 2048
NG = PX_PER_TEC // NB

_mesh = plsc.VectorSubcoreMesh(core_axis_name="c", subcore_axis_name="s")

_scratch = [
    pltpu.VMEM_SHARED((HW,), jnp.float32),
    pltpu.VMEM_SHARED((HW,), jnp.float32),
    pltpu.VMEM_SHARED((HW,), jnp.float32),
    pltpu.VMEM((CH,), jnp.float32),
    pltpu.VMEM((CH,), jnp.float32),
    pltpu.VMEM((CH,), jnp.float32),
    pltpu.VMEM((CH,), jnp.float32),
]
for _ in range(2):
    _scratch += [
        pltpu.VMEM((CH,), jnp.float32),
        pltpu.VMEM((CH,), jnp.float32),
        pltpu.VMEM((CH,), jnp.float32),
        pltpu.VMEM((CH,), jnp.int32),
        pltpu.VMEM((CH,), jnp.int32),
        pltpu.VMEM((CH,), jnp.int32),
        pltpu.VMEM((CH,), jnp.int32),
    ]
_scratch += [pltpu.VMEM((NB,), jnp.float32)]
for _ in range(2):
    _scratch += [
        pltpu.VMEM((NB,), jnp.float32),
        pltpu.VMEM((NB,), jnp.float32),
        pltpu.VMEM((NB,), jnp.float32),
        pltpu.VMEM((NB,), jnp.float32),
        pltpu.VMEM((NB,), jnp.float32),
    ]
_scratch += [pltpu.SemaphoreType.DMA] * 8


@functools.partial(
    pl.kernel,
    mesh=_mesh,
    out_type=jax.ShapeDtypeStruct((B, C, HW), jnp.float32),
    scratch_types=_scratch,
)
def _warp_kernel(in_hbm, out_hbm, accx, accy, accc, *rest):
    inbufs = [rest[0:2], rest[2:4]]
    cbufs = [rest[4 + 7 * p:4 + 7 * p + 7] for p in range(2)]
    zbuf = rest[18]
    nbufs = [rest[19 + 5 * p:19 + 5 * p + 5] for p in range(2)]
    sem_stream = rest[29]
    sem_load = rest[30:32]
    sem_store = rest[32:34]
    sem_zero = rest[34]
    sem_in = rest[35:37]
    cid = lax.axis_index("c")
    sid = lax.axis_index("s")
    px0 = sid * PX_PER_TEC
    lanes = lax.iota(jnp.int32, L)
    zeros16 = jnp.zeros((L,), jnp.float32)
    lim = jnp.float32(W - 1)

    def zfill(k, _):
        zbuf[pl.ds(k * L, L)] = zeros16
        return 0
    lax.fori_loop(0, NB // L, zfill, 0)

    def fire_zero(g):
        base = px0 + g * NB
        return [
            pltpu.async_copy(zbuf, accx.at[pl.ds(base, NB)], sem_zero),
            pltpu.async_copy(zbuf, accy.at[pl.ds(base, NB)], sem_zero),
            pltpu.async_copy(zbuf, accc.at[pl.ds(base, NB)], sem_zero),
        ]

    def fire_input(p, b, ci):
        fxb, fyb = inbufs[p]
        cbase = px0 + ci * CH
        return [
            pltpu.async_copy(in_hbm.at[b, 0, pl.ds(cbase, CH)], fxb,
                             sem_in[p]),
            pltpu.async_copy(in_hbm.at[b, 1, pl.ds(cbase, CH)], fyb,
                             sem_in[p]),
        ]

    def compute_chunk(p, ci):
        fxb, fyb = inbufs[p]
        vxb, vyb, wb, idx0, idx1, idx2, idx3 = cbufs[p]
        cbase = px0 + ci * CH

        def vec(kv, _):
            k = kv * L
            p_ = cbase + k + lanes
            row = lax.shift_right_logical(p_, 9)
            col = lax.bitwise_and(p_, W - 1)
            fxv = fxb[pl.ds(k, L)]
            fyv = fyb[pl.ds(k, L)]
            x2 = col.astype(jnp.float32) + fxv
            y2 = row.astype(jnp.float32) + fyv
            valid = ((x2 >= 0.0) & (y2 >= 0.0)
                     & (x2 <= lim) & (y2 <= lim))
            w1 = jnp.where(valid, jnp.float32(1.0), jnp.float32(0.0))
            ixl = jnp.clip(x2.astype(jnp.int32), 0, W - 1)
            iyt = jnp.clip(y2.astype(jnp.int32), 0, H - 1)
            ixr = jnp.minimum(ixl + 1, W - 1)
            iyb = jnp.minimum(iyt + 1, H - 1)
            vxb[pl.ds(k, L)] = -fxv * w1
            vyb[pl.ds(k, L)] = -fyv * w1
            wb[pl.ds(k, L)] = w1
            rt = iyt * W
            rb = iyb * W
            idx0[pl.ds(k, L)] = rt + ixl
            idx1[pl.ds(k, L)] = rt + ixr
            idx2[pl.ds(k, L)] = rb + ixl
            idx3[pl.ds(k, L)] = rb + ixr
            return 0
        lax.fori_loop(0, CH // L, vec, 0)

    def fire_chunk(p):
        vxb, vyb, wb, idx0, idx1, idx2, idx3 = cbufs[p]
        copies = []
        for ixr_ref in (idx0, idx1, idx2, idx3):
            copies.append(pltpu.async_copy(
                vxb, accx.at[ixr_ref], sem_stream, add=True))
            copies.append(pltpu.async_copy(
                vyb, accy.at[ixr_ref], sem_stream, add=True))
            copies.append(pltpu.async_copy(
                wb, accc.at[ixr_ref], sem_stream, add=True))
        return copies

    def fire_load(g, p):
        base = px0 + g * NB
        nbx, nby, nbc, _, _ = nbufs[p]
        return [
            pltpu.async_copy(accx.at[pl.ds(base, NB)], nbx, sem_load[p]),
            pltpu.async_copy(accy.at[pl.ds(base, NB)], nby, sem_load[p]),
            pltpu.async_copy(accc.at[pl.ds(base, NB)], nbc, sem_load[p]),
        ]

    def do_batch(b):
        inflight = None
        in_flt = [None, None]
        in_flt[0] = fire_input(0, b, 0)
        for ci in range(NCHUNK):
            p = ci % 2
            if ci + 1 < NCHUNK:
                in_flt[1 - p] = fire_input(1 - p, b, ci + 1)
            for cp in in_flt[p]:
                cp.wait()
            compute_chunk(p, ci)
            if inflight is not None:
                for cp in inflight:
                    cp.wait()
            inflight = fire_chunk(p)
        for cp in inflight:
            cp.wait()
        plsc.subcore_barrier()

        zero_copies = []
        store_copies = [None, None]
        loads = [None, None]
        loads[0] = fire_load(0, 0)
        for g in range(NG):
            p = g % 2
            if g + 1 < NG:
                loads[1 - p] = fire_load(g + 1, 1 - p)
            for cp in loads[p]:
                cp.wait()
            zero_copies += fire_zero(g)
            nbx, nby, nbc, oxb, oyb = nbufs[p]
            if store_copies[p] is not None:
                for cp in store_copies[p]:
                    cp.wait()

            def nv(kv, _):
                k = kv * L
                sx = nbx[pl.ds(k, L)]
                sy = nby[pl.ds(k, L)]
                cc = nbc[pl.ds(k, L)]
                r = jnp.float32(1.0) / jnp.where(cc > 0.0, cc,
                                                 jnp.float32(1.0))
                oxb[pl.ds(k, L)] = sx * r
                oyb[pl.ds(k, L)] = sy * r
                return 0
            lax.fori_loop(0, NB // L, nv, 0)
            base = px0 + g * NB
            store_copies[p] = [
                pltpu.async_copy(oxb, out_hbm.at[b, 0, pl.ds(base, NB)],
                                 sem_store[p]),
                pltpu.async_copy(oyb, out_hbm.at[b, 1, pl.ds(base, NB)],
                                 sem_store[p]),
            ]
        for sc_list in store_copies:
            if sc_list is not None:
                for cp in sc_list:
                    cp.wait()
        for cp in zero_copies:
            cp.wait()
        plsc.subcore_barrier()

    prol = []
    for g in range(NG):
        prol += fire_zero(g)
    for cp in prol:
        cp.wait()
    plsc.subcore_barrier()

    def batch_loop(bi, _):
        do_batch(cid * BPC + bi)
        return 0
    lax.fori_loop(0, BPC, batch_loop, 0)


def kernel(input1):
    flat = input1.reshape(B, C, HW)
    out = _warp_kernel(flat)
    return out.reshape(B, C, H, W)
